# baseline (speedup 1.0000x reference)
import functools

import jax
import jax.numpy as jnp
from jax import lax
from jax.experimental import pallas as pl
from jax.experimental.pallas import tpu as pltpu
from jax.experimental.pallas import tpu_sc as plsc

S1, S2 = 8192, 4096
R = 16
K = 8

_info = plsc.get_sparse_core_info()
_NC, _NS = _info.num_cores, _info.num_subcores
_NW = _NC * _NS

_N1 = S1 * K
_N2 = S2 * K
_C1 = _N1 // _NW
_C2 = _N2 // _NW
_W1 = S1 // _NW
_W2 = S2 // _NW


def _interleave(kt_v, i_v, n_out, w):
    def step(t, carry):
        base = t * R
        jv = base + lax.iota(jnp.int32, R)
        src = (jv & (K - 1)) * w + lax.shift_right_logical(jv, 3)
        i_v[pl.ds(base, R)] = plsc.load_gather(kt_v, [src])
        return carry

    lax.fori_loop(0, n_out // R, step, 0)


def _sc_gather_body(i1_hbm, i2_hbm, m1_hbm, m2_hbm, o1_hbm, o2_hbm,
                    m1_sh, m2_sh, k1_v, k2_v, i1_v, r1_v, i2_v, r2_v,
                    sem, sem2, sem3):
    sid = lax.axis_index("s")
    wid = sid * _NC + lax.axis_index("c")
    b1 = wid * _C1
    b2 = wid * _C2
    wb1 = wid * _W1
    wb2 = wid * _W2

    @pl.when(sid == 0)
    def _():
        pltpu.sync_copy(m1_hbm, m1_sh)

    @pl.when(sid == 1)
    def _():
        pltpu.sync_copy(m2_hbm, m2_sh)

    cps = []
    for k in range(K):
        cps.append(pltpu.async_copy(
            i1_hbm.at[pl.ds(k * S1 + wb1, _W1)],
            k1_v.at[pl.ds(k * _W1, _W1)], sem3))
    for k in range(K):
        cps.append(pltpu.async_copy(
            i2_hbm.at[pl.ds(k * S2 + wb2, _W2)],
            k2_v.at[pl.ds(k * _W2, _W2)], sem3))
    for cp in cps:
        cp.wait()

    _interleave(k1_v, i1_v, _C1, _W1)
    plsc.subcore_barrier()
    h1 = _C1 // 2
    cp1a = pltpu.async_copy(m1_sh.at[i1_v.at[pl.ds(0, h1)]],
                            r1_v.at[pl.ds(0, h1), :], sem)
    cp1b = pltpu.async_copy(m1_sh.at[i1_v.at[pl.ds(h1, h1)]],
                            r1_v.at[pl.ds(h1, h1), :], sem2)
    _interleave(k2_v, i2_v, _C2, _W2)
    cp1a.wait()
    pltpu.sync_copy(r1_v.at[pl.ds(0, h1), :], o1_hbm.at[pl.ds(b1, h1)])
    cp2 = pltpu.async_copy(m2_sh.at[i2_v], r2_v, sem3)
    cp1b.wait()
    pltpu.sync_copy(r1_v.at[pl.ds(h1, h1), :],
                    o1_hbm.at[pl.ds(b1 + h1, h1)])
    cp2.wait()
    pltpu.sync_copy(r2_v, o2_hbm.at[pl.ds(b2, _C2)])


_sc_gather = functools.partial(
    pl.kernel,
    out_type=(
        jax.ShapeDtypeStruct((_N1, R), jnp.bfloat16),
        jax.ShapeDtypeStruct((_N2, R), jnp.bfloat16),
    ),
    mesh=plsc.VectorSubcoreMesh(core_axis_name="c", subcore_axis_name="s"),
    scratch_types=[
        pltpu.VMEM_SHARED((S1, R), jnp.bfloat16),
        pltpu.VMEM_SHARED((S2, R), jnp.bfloat16),
        pltpu.VMEM((_C1,), jnp.int32),
        pltpu.VMEM((_C2,), jnp.int32),
        pltpu.VMEM((_C1,), jnp.int32),
        pltpu.VMEM((_C1, R), jnp.bfloat16),
        pltpu.VMEM((_C2,), jnp.int32),
        pltpu.VMEM((_C2, R), jnp.bfloat16),
        pltpu.SemaphoreType.DMA,
        pltpu.SemaphoreType.DMA,
        pltpu.SemaphoreType.DMA,
    ],
    compiler_params=pltpu.CompilerParams(use_tc_tiling_on_sc=False,
                                         needs_layout_passes=False),
)(_sc_gather_body)


_BI = 1024


def _main_body(w2_ref, r2_ref, g_ref, w1_ref, r1_ref, out_ref, b8_ref):
    i = pl.program_id(0)

    @pl.when(i == 0)
    def _():
        r2 = r2_ref[...].astype(jnp.float32)
        acc = w2_ref[:, 0:R] * r2[:, 0:R]
        for k in range(1, K):
            acc = acc + (w2_ref[:, k * R:(k + 1) * R]
                         * r2[:, k * R:(k + 1) * R])
        bt = lax.dot_general(
            g_ref[...], acc, (((1,), (1,)), ((), ())),
            preferred_element_type=jnp.float32)
        for k in range(K):
            b8_ref[k * R:(k + 1) * R, :] = bt

    p = w1_ref[...] * r1_ref[...].astype(jnp.float32)
    out_ref[...] = jnp.dot(p, b8_ref[...],
                           preferred_element_type=jnp.float32)


def kernel(x, M1, M2, G, idx1, idx2, dist1, dist2):
    del x
    r1, r2 = _sc_gather(idx1.T.reshape(-1), idx2.T.reshape(-1),
                        M1.astype(jnp.bfloat16), M2.astype(jnp.bfloat16))
    r1f = r1.reshape(S1, K * R)
    r2f = r2.reshape(S2, K * R)
    we1 = jnp.repeat(dist1, R, axis=1)
    we2 = jnp.repeat(dist2, R, axis=1)

    out = pl.pallas_call(
        _main_body,
        grid=(S1 // _BI,),
        in_specs=[
            pl.BlockSpec((S2, K * R), lambda i: (0, 0)),
            pl.BlockSpec((S2, K * R), lambda i: (0, 0)),
            pl.BlockSpec((R, R), lambda i: (0, 0)),
            pl.BlockSpec((_BI, K * R), lambda i: (i, 0)),
            pl.BlockSpec((_BI, K * R), lambda i: (i, 0)),
        ],
        out_specs=pl.BlockSpec((_BI, S2), lambda i: (i, 0)),
        out_shape=jax.ShapeDtypeStruct((S1, S2), jnp.float32),
        scratch_shapes=[pltpu.VMEM((K * R, S2), jnp.float32)],
        compiler_params=pltpu.CompilerParams(
            dimension_semantics=("arbitrary",)),
    )(we2, r2f, G, we1, r1f)
    return out


# speedup vs baseline: 1.1165x; 1.1165x over previous
import functools

import jax
import jax.numpy as jnp
from jax import lax
from jax.experimental import pallas as pl
from jax.experimental.pallas import tpu as pltpu
from jax.experimental.pallas import tpu_sc as plsc

S1, S2 = 8192, 4096
R = 16
K = 8

_info = plsc.get_sparse_core_info()
_NC, _NS = _info.num_cores, _info.num_subcores
_NW = _NC * _NS

_N1 = S1 * K
_N2 = S2 * K
_C1 = _N1 // _NW
_C2 = _N2 // _NW
_W1 = S1 // _NW
_W2 = S2 // _NW


def _interleave(kt_v, i_v, n_out, w):
    def step(t, carry):
        base = t * R
        jv = base + lax.iota(jnp.int32, R)
        src = (jv & (K - 1)) * w + lax.shift_right_logical(jv, 3)
        i_v[pl.ds(base, R)] = plsc.load_gather(kt_v, [src])
        return carry

    lax.fori_loop(0, n_out // R, step, 0)


def _sc_gather_body(i1_hbm, i2_hbm, m1_hbm, m2_hbm, o1_hbm, o2_hbm,
                    m1_sh, m2_sh, k1_v, k2_v, i1_v, r1_v, i2_v, r2_v,
                    sem, sem2, sem3):
    sid = lax.axis_index("s")
    wid = sid * _NC + lax.axis_index("c")
    b1 = wid * _C1
    b2 = wid * _C2
    wb1 = wid * _W1
    wb2 = wid * _W2

    @pl.when(sid == 0)
    def _():
        pltpu.sync_copy(m1_hbm, m1_sh)

    @pl.when(sid == 1)
    def _():
        pltpu.sync_copy(m2_hbm, m2_sh)

    cps = []
    for k in range(K):
        cps.append(pltpu.async_copy(
            i1_hbm.at[pl.ds(k * S1 + wb1, _W1)],
            k1_v.at[pl.ds(k * _W1, _W1)], sem3))
    for k in range(K):
        cps.append(pltpu.async_copy(
            i2_hbm.at[pl.ds(k * S2 + wb2, _W2)],
            k2_v.at[pl.ds(k * _W2, _W2)], sem3))
    for cp in cps:
        cp.wait()

    _interleave(k1_v, i1_v, _C1, _W1)
    plsc.subcore_barrier()
    h1 = _C1 // 2
    cp1a = pltpu.async_copy(m1_sh.at[i1_v.at[pl.ds(0, h1)]],
                            r1_v.at[pl.ds(0, h1), :], sem)
    cp1b = pltpu.async_copy(m1_sh.at[i1_v.at[pl.ds(h1, h1)]],
                            r1_v.at[pl.ds(h1, h1), :], sem2)
    _interleave(k2_v, i2_v, _C2, _W2)
    cp1a.wait()
    pltpu.sync_copy(r1_v.at[pl.ds(0, h1), :], o1_hbm.at[pl.ds(b1, h1)])
    cp2 = pltpu.async_copy(m2_sh.at[i2_v], r2_v, sem3)
    cp1b.wait()
    pltpu.sync_copy(r1_v.at[pl.ds(h1, h1), :],
                    o1_hbm.at[pl.ds(b1 + h1, h1)])
    cp2.wait()
    pltpu.sync_copy(r2_v, o2_hbm.at[pl.ds(b2, _C2)])


_sc_gather = functools.partial(
    pl.kernel,
    out_type=(
        jax.ShapeDtypeStruct((_N1, R), jnp.float32),
        jax.ShapeDtypeStruct((_N2, R), jnp.float32),
    ),
    mesh=plsc.VectorSubcoreMesh(core_axis_name="c", subcore_axis_name="s"),
    scratch_types=[
        pltpu.VMEM_SHARED((S1, R), jnp.float32),
        pltpu.VMEM_SHARED((S2, R), jnp.float32),
        pltpu.VMEM((_C1,), jnp.int32),
        pltpu.VMEM((_C2,), jnp.int32),
        pltpu.VMEM((_C1,), jnp.int32),
        pltpu.VMEM((_C1, R), jnp.float32),
        pltpu.VMEM((_C2,), jnp.int32),
        pltpu.VMEM((_C2, R), jnp.float32),
        pltpu.SemaphoreType.DMA,
        pltpu.SemaphoreType.DMA,
        pltpu.SemaphoreType.DMA,
    ],
    compiler_params=pltpu.CompilerParams(use_tc_tiling_on_sc=False,
                                         needs_layout_passes=False),
)(_sc_gather_body)


_BI = 1024


def _main_body(w2_ref, r2_ref, g_ref, w1_ref, r1_ref, out_ref, b8_ref):
    i = pl.program_id(0)

    @pl.when(i == 0)
    def _():
        acc = w2_ref[:, 0:R] * r2_ref[:, 0:R]
        for k in range(1, K):
            acc = acc + (w2_ref[:, k * R:(k + 1) * R]
                         * r2_ref[:, k * R:(k + 1) * R])
        bt = lax.dot_general(
            g_ref[...], acc, (((1,), (1,)), ((), ())),
            preferred_element_type=jnp.float32)
        for k in range(K):
            b8_ref[k * R:(k + 1) * R, :] = bt

    p = w1_ref[...] * r1_ref[...]
    out_ref[...] = jnp.dot(p, b8_ref[...],
                           preferred_element_type=jnp.float32)


def kernel(x, M1, M2, G, idx1, idx2, dist1, dist2):
    del x
    r1, r2 = _sc_gather(idx1.T.reshape(-1), idx2.T.reshape(-1), M1, M2)
    r1f = r1.reshape(S1, K * R)
    r2f = r2.reshape(S2, K * R)
    we1 = jnp.repeat(dist1, R, axis=1)
    we2 = jnp.repeat(dist2, R, axis=1)

    out = pl.pallas_call(
        _main_body,
        grid=(S1 // _BI,),
        in_specs=[
            pl.BlockSpec((S2, K * R), lambda i: (0, 0)),
            pl.BlockSpec((S2, K * R), lambda i: (0, 0)),
            pl.BlockSpec((R, R), lambda i: (0, 0)),
            pl.BlockSpec((_BI, K * R), lambda i: (i, 0)),
            pl.BlockSpec((_BI, K * R), lambda i: (i, 0)),
        ],
        out_specs=pl.BlockSpec((_BI, S2), lambda i: (i, 0)),
        out_shape=jax.ShapeDtypeStruct((S1, S2), jnp.float32),
        scratch_shapes=[pltpu.VMEM((K * R, S2), jnp.float32)],
        compiler_params=pltpu.CompilerParams(
            dimension_semantics=("arbitrary",)),
    )(we2, r2f, G, we1, r1f)
    return out


# speedup vs baseline: 1.1173x; 1.0007x over previous
import functools

import jax
import jax.numpy as jnp
from jax import lax
from jax.experimental import pallas as pl
from jax.experimental.pallas import tpu as pltpu
from jax.experimental.pallas import tpu_sc as plsc

S1, S2 = 8192, 4096
R = 16
K = 8

_info = plsc.get_sparse_core_info()
_NC, _NS = _info.num_cores, _info.num_subcores
_NW = _NC * _NS

_N1 = S1 * K
_N2 = S2 * K
_C1 = _N1 // _NW
_C2 = _N2 // _NW
_W1 = S1 // _NW
_W2 = S2 // _NW


def _interleave(kt_v, i_v, n_out, w):
    def step(t, carry):
        base = t * R
        jv = base + lax.iota(jnp.int32, R)
        src = (jv & (K - 1)) * w + lax.shift_right_logical(jv, 3)
        i_v[pl.ds(base, R)] = plsc.load_gather(kt_v, [src])
        return carry

    lax.fori_loop(0, n_out // R, step, 0, unroll=4)


def _sc_gather_body(i1_hbm, i2_hbm, m1_hbm, m2_hbm, o1_hbm, o2_hbm,
                    m1_sh, m2_sh, k1_v, k2_v, i1_v, r1_v, i2_v, r2_v,
                    sem, sem2, sem3):
    sid = lax.axis_index("s")
    wid = sid * _NC + lax.axis_index("c")
    b1 = wid * _C1
    b2 = wid * _C2
    wb1 = wid * _W1
    wb2 = wid * _W2

    @pl.when(sid == 0)
    def _():
        pltpu.sync_copy(m1_hbm, m1_sh)

    @pl.when(sid == 1)
    def _():
        pltpu.sync_copy(m2_hbm, m2_sh)

    cps = []
    for k in range(K):
        cps.append(pltpu.async_copy(
            i1_hbm.at[pl.ds(k * S1 + wb1, _W1)],
            k1_v.at[pl.ds(k * _W1, _W1)], sem3))
    for k in range(K):
        cps.append(pltpu.async_copy(
            i2_hbm.at[pl.ds(k * S2 + wb2, _W2)],
            k2_v.at[pl.ds(k * _W2, _W2)], sem3))
    for cp in cps:
        cp.wait()

    _interleave(k1_v, i1_v, _C1, _W1)
    plsc.subcore_barrier()
    h1 = _C1 // 2
    cp1a = pltpu.async_copy(m1_sh.at[i1_v.at[pl.ds(0, h1)]],
                            r1_v.at[pl.ds(0, h1), :], sem)
    cp1b = pltpu.async_copy(m1_sh.at[i1_v.at[pl.ds(h1, h1)]],
                            r1_v.at[pl.ds(h1, h1), :], sem2)
    _interleave(k2_v, i2_v, _C2, _W2)
    cp1a.wait()
    pltpu.sync_copy(r1_v.at[pl.ds(0, h1), :], o1_hbm.at[pl.ds(b1, h1)])
    cp2 = pltpu.async_copy(m2_sh.at[i2_v], r2_v, sem3)
    cp1b.wait()
    pltpu.sync_copy(r1_v.at[pl.ds(h1, h1), :],
                    o1_hbm.at[pl.ds(b1 + h1, h1)])
    cp2.wait()
    pltpu.sync_copy(r2_v, o2_hbm.at[pl.ds(b2, _C2)])


_sc_gather = functools.partial(
    pl.kernel,
    out_type=(
        jax.ShapeDtypeStruct((_N1, R), jnp.float32),
        jax.ShapeDtypeStruct((_N2, R), jnp.float32),
    ),
    mesh=plsc.VectorSubcoreMesh(core_axis_name="c", subcore_axis_name="s"),
    scratch_types=[
        pltpu.VMEM_SHARED((S1, R), jnp.float32),
        pltpu.VMEM_SHARED((S2, R), jnp.float32),
        pltpu.VMEM((_C1,), jnp.int32),
        pltpu.VMEM((_C2,), jnp.int32),
        pltpu.VMEM((_C1,), jnp.int32),
        pltpu.VMEM((_C1, R), jnp.float32),
        pltpu.VMEM((_C2,), jnp.int32),
        pltpu.VMEM((_C2, R), jnp.float32),
        pltpu.SemaphoreType.DMA,
        pltpu.SemaphoreType.DMA,
        pltpu.SemaphoreType.DMA,
    ],
    compiler_params=pltpu.CompilerParams(use_tc_tiling_on_sc=False,
                                         needs_layout_passes=False),
)(_sc_gather_body)


_BI = 1024


def _main_body(w2_ref, r2_ref, g_ref, w1_ref, r1_ref, out_ref, b8_ref):
    i = pl.program_id(0)

    @pl.when(i == 0)
    def _():
        acc = w2_ref[:, 0:R] * r2_ref[:, 0:R]
        for k in range(1, K):
            acc = acc + (w2_ref[:, k * R:(k + 1) * R]
                         * r2_ref[:, k * R:(k + 1) * R])
        bt = lax.dot_general(
            g_ref[...], acc, (((1,), (1,)), ((), ())),
            preferred_element_type=jnp.float32)
        for k in range(K):
            b8_ref[k * R:(k + 1) * R, :] = bt

    p = w1_ref[...] * r1_ref[...]
    out_ref[...] = jnp.dot(p, b8_ref[...],
                           preferred_element_type=jnp.float32)


def kernel(x, M1, M2, G, idx1, idx2, dist1, dist2):
    del x
    r1, r2 = _sc_gather(idx1.T.reshape(-1), idx2.T.reshape(-1), M1, M2)
    r1f = r1.reshape(S1, K * R)
    r2f = r2.reshape(S2, K * R)
    we1 = jnp.repeat(dist1, R, axis=1)
    we2 = jnp.repeat(dist2, R, axis=1)

    out = pl.pallas_call(
        _main_body,
        grid=(S1 // _BI,),
        in_specs=[
            pl.BlockSpec((S2, K * R), lambda i: (0, 0)),
            pl.BlockSpec((S2, K * R), lambda i: (0, 0)),
            pl.BlockSpec((R, R), lambda i: (0, 0)),
            pl.BlockSpec((_BI, K * R), lambda i: (i, 0)),
            pl.BlockSpec((_BI, K * R), lambda i: (i, 0)),
        ],
        out_specs=pl.BlockSpec((_BI, S2), lambda i: (i, 0)),
        out_shape=jax.ShapeDtypeStruct((S1, S2), jnp.float32),
        scratch_shapes=[pltpu.VMEM((K * R, S2), jnp.float32)],
        compiler_params=pltpu.CompilerParams(
            dimension_semantics=("arbitrary",)),
    )(we2, r2f, G, we1, r1f)
    return out
